# baseline (speedup 1.0000x reference)
import functools

import jax
import jax.numpy as jnp
from jax.experimental import pallas as pl
from jax.experimental.pallas import tpu as pltpu

B = 8192
IN_DIM = 768
DIM = 2048
LATENT = 64
K = 1024
BLK = 1024

_HI = jax.lax.Precision.HIGHEST
_DEF = jax.lax.Precision.DEFAULT


def _dot(a, b, precision, dims=None):
    dn = dims if dims is not None else (((1,), (0,)), ((), ()))
    return jax.lax.dot_general(a, b, dn, precision=precision,
                               preferred_element_type=jnp.float32)


def _body(x_ref, We1_ref, be1_ref, Wmu_ref, bmu_ref, cb_ref, cbn_ref,
          Wd1_ref, bd1_ref, Wm_ref, bm_ref, Wl_ref, bl_ref,
          mean_ref, lv_ref, ze_ref, zq_ref):
    x = x_ref[...]
    h = jnp.maximum(_dot(x, We1_ref[...], _DEF) + be1_ref[...], 0.0)
    z = _dot(h, Wmu_ref[...], _DEF) + bmu_ref[...]
    ze_ref[...] = z
    s = _dot(z, cb_ref[...], _DEF, dims=(((1,), (1,)), ((), ())))
    dist = (jnp.sum(z * z, axis=1, keepdims=True) + cbn_ref[...]) - 2.0 * s
    m = jnp.min(dist, axis=1, keepdims=True)
    iota = jax.lax.broadcasted_iota(jnp.int32, dist.shape, 1)
    idx = jnp.min(jnp.where(dist <= m, iota, K), axis=1, keepdims=True)
    onehot = (iota == idx).astype(jnp.float32)
    zq = _dot(onehot, cb_ref[...], _DEF)
    zq_ref[...] = zq
    hd = jnp.maximum(_dot(zq.astype(jnp.bfloat16), Wd1_ref[...], _DEF)
                     + bd1_ref[...], 0.0)
    hdb = hd.astype(jnp.bfloat16)
    mean_ref[...] = _dot(hdb, Wm_ref[...], _DEF) + bm_ref[...]
    lv_ref[...] = _dot(hdb, Wl_ref[...], _DEF) + bl_ref[...]


@jax.jit
def kernel(x, We1, be1, Wmu, bmu, Wlv, blv, codebook, Wd1, bd1, Wm, bm, Wl, bl):
    del Wlv, blv
    cbn = jnp.sum(codebook * codebook, axis=1)[None, :]
    grid = (B // BLK,)
    row = lambda i: (i, 0)
    rep = lambda i: (0, 0)
    full = lambda s: pl.BlockSpec(s, rep)
    out = pl.pallas_call(
        _body,
        grid=grid,
        compiler_params=pltpu.CompilerParams(
            dimension_semantics=("parallel",)),
        in_specs=[
            pl.BlockSpec((BLK, IN_DIM), row),
            full((IN_DIM, DIM)), full((1, DIM)),
            full((DIM, LATENT)), full((1, LATENT)),
            full((K, LATENT)), full((1, K)),
            full((LATENT, DIM)), full((1, DIM)),
            full((DIM, IN_DIM)), full((1, IN_DIM)),
            full((DIM, IN_DIM)), full((1, IN_DIM)),
        ],
        out_specs=[
            pl.BlockSpec((BLK, IN_DIM), row),
            pl.BlockSpec((BLK, IN_DIM), row),
            pl.BlockSpec((BLK, LATENT), row),
            pl.BlockSpec((BLK, LATENT), row),
        ],
        out_shape=[
            jax.ShapeDtypeStruct((B, IN_DIM), jnp.float32),
            jax.ShapeDtypeStruct((B, IN_DIM), jnp.float32),
            jax.ShapeDtypeStruct((B, LATENT), jnp.float32),
            jax.ShapeDtypeStruct((B, LATENT), jnp.float32),
        ],
    )(x, We1, be1[None, :], Wmu, bmu[None, :], codebook, cbn,
      Wd1.astype(jnp.bfloat16), bd1[None, :],
      Wm.astype(jnp.bfloat16), bm[None, :],
      Wl.astype(jnp.bfloat16), bl[None, :])
    mean, log_var, z_e_x, z_q_x = out
    return (mean, log_var, z_e_x, z_q_x)


# speedup vs baseline: 1.0137x; 1.0137x over previous
import functools

import jax
import jax.numpy as jnp
from jax.experimental import pallas as pl
from jax.experimental.pallas import tpu as pltpu

B = 8192
IN_DIM = 768
DIM = 2048
LATENT = 64
K = 1024
BLK = 1024

_HI = jax.lax.Precision.HIGHEST
_DEF = jax.lax.Precision.DEFAULT


def _dot(a, b, precision, dims=None):
    dn = dims if dims is not None else (((1,), (0,)), ((), ()))
    return jax.lax.dot_general(a, b, dn, precision=precision,
                               preferred_element_type=jnp.float32)


def _body(x_ref, We1_ref, be1_ref, Wmu_ref, bmu_ref, cb_ref, cbn_ref,
          Wd1_ref, bd1_ref, Wm_ref, bm_ref, Wl_ref, bl_ref,
          mean_ref, lv_ref, ze_ref, zq_ref):
    x = x_ref[...]
    h = jnp.maximum(_dot(x, We1_ref[...], _DEF) + be1_ref[...], 0.0)
    z = _dot(h, Wmu_ref[...], _DEF) + bmu_ref[...]
    ze_ref[...] = z
    s = _dot(z, cb_ref[...], _DEF, dims=(((1,), (1,)), ((), ())))
    dist = (jnp.sum(z * z, axis=1, keepdims=True) + cbn_ref[...]) - 2.0 * s
    m = jnp.min(dist, axis=1, keepdims=True)
    iota = jax.lax.broadcasted_iota(jnp.int32, dist.shape, 1)
    idx = jnp.min(jnp.where(dist <= m, iota, K), axis=1, keepdims=True)
    onehot = (iota == idx).astype(jnp.float32)
    zq = _dot(onehot, cb_ref[...], _DEF)
    zq_ref[...] = zq
    hd = jnp.maximum(_dot(zq, Wd1_ref[...], _DEF) + bd1_ref[...], 0.0)
    hdb = hd.astype(jnp.bfloat16)
    mean_ref[...] = _dot(hdb, Wm_ref[...], _DEF) + bm_ref[...]
    lv_ref[...] = _dot(hdb, Wl_ref[...], _DEF) + bl_ref[...]


@jax.jit
def kernel(x, We1, be1, Wmu, bmu, Wlv, blv, codebook, Wd1, bd1, Wm, bm, Wl, bl):
    del Wlv, blv
    cbn = jnp.sum(codebook * codebook, axis=1)[None, :]
    grid = (B // BLK,)
    row = lambda i: (i, 0)
    rep = lambda i: (0, 0)
    full = lambda s: pl.BlockSpec(s, rep)
    out = pl.pallas_call(
        _body,
        grid=grid,
        compiler_params=pltpu.CompilerParams(
            dimension_semantics=("parallel",)),
        in_specs=[
            pl.BlockSpec((BLK, IN_DIM), row),
            full((IN_DIM, DIM)), full((1, DIM)),
            full((DIM, LATENT)), full((1, LATENT)),
            full((K, LATENT)), full((1, K)),
            full((LATENT, DIM)), full((1, DIM)),
            full((DIM, IN_DIM)), full((1, IN_DIM)),
            full((DIM, IN_DIM)), full((1, IN_DIM)),
        ],
        out_specs=[
            pl.BlockSpec((BLK, IN_DIM), row),
            pl.BlockSpec((BLK, IN_DIM), row),
            pl.BlockSpec((BLK, LATENT), row),
            pl.BlockSpec((BLK, LATENT), row),
        ],
        out_shape=[
            jax.ShapeDtypeStruct((B, IN_DIM), jnp.float32),
            jax.ShapeDtypeStruct((B, IN_DIM), jnp.float32),
            jax.ShapeDtypeStruct((B, LATENT), jnp.float32),
            jax.ShapeDtypeStruct((B, LATENT), jnp.float32),
        ],
    )(x, We1, be1[None, :], Wmu, bmu[None, :], codebook, cbn,
      Wd1, bd1[None, :],
      Wm.astype(jnp.bfloat16), bm[None, :],
      Wl.astype(jnp.bfloat16), bl[None, :])
    mean, log_var, z_e_x, z_q_x = out
    return (mean, log_var, z_e_x, z_q_x)


# speedup vs baseline: 1.0486x; 1.0345x over previous
import functools

import jax
import jax.numpy as jnp
from jax.experimental import pallas as pl
from jax.experimental.pallas import tpu as pltpu

B = 8192
IN_DIM = 768
DIM = 2048
LATENT = 64
K = 1024
BLK = 1024

_HI = jax.lax.Precision.HIGHEST
_DEF = jax.lax.Precision.DEFAULT


def _dot(a, b, precision, dims=None):
    dn = dims if dims is not None else (((1,), (0,)), ((), ()))
    return jax.lax.dot_general(a, b, dn, precision=precision,
                               preferred_element_type=jnp.float32)


def _body(x_ref, We1_ref, be1_ref, Wmu_ref, bmu_ref, cb_ref, cbn_ref,
          Wd1_ref, bd1_ref, Wm_ref, bm_ref, Wl_ref, bl_ref,
          mean_ref, lv_ref, ze_ref, zq_ref):
    x = x_ref[...]
    h = jnp.maximum(_dot(x, We1_ref[...], _DEF) + be1_ref[...], 0.0)
    z = _dot(h, Wmu_ref[...], _DEF) + bmu_ref[...]
    ze_ref[...] = z
    s = _dot(z, cb_ref[...], _DEF, dims=(((1,), (1,)), ((), ())))
    dist = (jnp.sum(z * z, axis=1, keepdims=True) + cbn_ref[...]) - 2.0 * s
    m = jnp.min(dist, axis=1, keepdims=True)
    iota = jax.lax.broadcasted_iota(jnp.int32, dist.shape, 1)
    idx = jnp.min(jnp.where(dist <= m, iota, K), axis=1, keepdims=True)
    onehot = (iota == idx).astype(jnp.float32)
    zq = _dot(onehot, cb_ref[...], _DEF)
    zq_ref[...] = zq
    hd = jnp.maximum(_dot(zq, Wd1_ref[...], _DEF) + bd1_ref[...], 0.0)
    hdb = hd.astype(jnp.bfloat16)
    mean_ref[...] = _dot(hdb, Wm_ref[...].astype(jnp.bfloat16), _DEF) + bm_ref[...]
    lv_ref[...] = _dot(hdb, Wl_ref[...].astype(jnp.bfloat16), _DEF) + bl_ref[...]


@jax.jit
def kernel(x, We1, be1, Wmu, bmu, Wlv, blv, codebook, Wd1, bd1, Wm, bm, Wl, bl):
    del Wlv, blv
    cbn = jnp.sum(codebook * codebook, axis=1)[None, :]
    grid = (B // BLK,)
    row = lambda i: (i, 0)
    rep = lambda i: (0, 0)
    full = lambda s: pl.BlockSpec(s, rep)
    out = pl.pallas_call(
        _body,
        grid=grid,
        compiler_params=pltpu.CompilerParams(
            dimension_semantics=("parallel",)),
        in_specs=[
            pl.BlockSpec((BLK, IN_DIM), row),
            full((IN_DIM, DIM)), full((1, DIM)),
            full((DIM, LATENT)), full((1, LATENT)),
            full((K, LATENT)), full((1, K)),
            full((LATENT, DIM)), full((1, DIM)),
            full((DIM, IN_DIM)), full((1, IN_DIM)),
            full((DIM, IN_DIM)), full((1, IN_DIM)),
        ],
        out_specs=[
            pl.BlockSpec((BLK, IN_DIM), row),
            pl.BlockSpec((BLK, IN_DIM), row),
            pl.BlockSpec((BLK, LATENT), row),
            pl.BlockSpec((BLK, LATENT), row),
        ],
        out_shape=[
            jax.ShapeDtypeStruct((B, IN_DIM), jnp.float32),
            jax.ShapeDtypeStruct((B, IN_DIM), jnp.float32),
            jax.ShapeDtypeStruct((B, LATENT), jnp.float32),
            jax.ShapeDtypeStruct((B, LATENT), jnp.float32),
        ],
    )(x, We1, be1[None, :], Wmu, bmu[None, :], codebook, cbn,
      Wd1, bd1[None, :], Wm, bm[None, :], Wl, bl[None, :])
    mean, log_var, z_e_x, z_q_x = out
    return (mean, log_var, z_e_x, z_q_x)
